# Initial kernel scaffold; baseline (speedup 1.0000x reference)
import jax
import jax.numpy as jnp
from jax.experimental import pallas as pl


def kernel(x, emb_weight, lora_A, lora_B):
    raise NotImplementedError("write your pallas kernel here")


# speedup vs baseline: 7.8978x; 7.8978x over previous
import functools

import jax
import jax.numpy as jnp
from jax import lax
from jax.experimental import pallas as pl
from jax.experimental.pallas import tpu as pltpu
from jax.experimental.pallas import tpu_sc as plsc

NUM_EMB = 100000
DIM = 64
RANK = 8
SCALE = 2.0


FUSE_ROWS = 2000


def _fuse_body(a_ref, emb_ref, b_ref, out_ref):
    delta = lax.dot_general(
        a_ref[...], b_ref[...],
        dimension_numbers=(((1,), (1,)), ((), ())),
        preferred_element_type=jnp.float32,
    )
    out_ref[...] = emb_ref[...] + SCALE * delta


def _fused_table(emb_weight, lora_At, lora_B):
    grid = NUM_EMB // FUSE_ROWS
    return pl.pallas_call(
        _fuse_body,
        grid=(grid,),
        in_specs=[
            pl.BlockSpec((FUSE_ROWS, RANK), lambda i: (i, 0)),
            pl.BlockSpec((FUSE_ROWS, DIM), lambda i: (i, 0)),
            pl.BlockSpec((DIM, RANK), lambda i: (0, 0)),
        ],
        out_specs=pl.BlockSpec((FUSE_ROWS, DIM), lambda i: (i, 0)),
        out_shape=jax.ShapeDtypeStruct((NUM_EMB, DIM), jnp.float32),
    )(lora_At, emb_weight, lora_B)


NUM_CORES = 2
NUM_SUBCORES = 16
NW = NUM_CORES * NUM_SUBCORES
B_TOTAL = 4096 * 200
B_PER_W = B_TOTAL // NW
CHUNK = 128
GROUP = 8
ROWS = CHUNK * GROUP
N_GROUPS = B_PER_W // ROWS
N_CHUNKS = B_PER_W // CHUNK


def _gather_body(table_hbm, idx_hbm, out_hbm, idx_v, rows_v, sem):
    wid = lax.axis_index("s") * NUM_CORES + lax.axis_index("c")
    pltpu.sync_copy(idx_hbm.at[wid], idx_v)

    def flush(g, carry):
        copies = []
        for j in range(GROUP):
            c = g * GROUP + j
            copies.append(
                pltpu.async_copy(
                    table_hbm.at[idx_v.at[c]],
                    rows_v.at[pl.ds(j * CHUNK, CHUNK)],
                    sem,
                )
            )
        for cp in copies:
            cp.wait()
        pltpu.sync_copy(rows_v, out_hbm.at[wid, g])
        return carry

    lax.fori_loop(0, N_GROUPS, flush, 0)


def _gather(table, idx):
    mesh = plsc.VectorSubcoreMesh(core_axis_name="c", subcore_axis_name="s")
    run = functools.partial(
        pl.kernel,
        mesh=mesh,
        out_type=jax.ShapeDtypeStruct((NW, N_GROUPS, ROWS, DIM), jnp.float32),
        scratch_types=[
            pltpu.VMEM((N_CHUNKS, CHUNK), jnp.int32),
            pltpu.VMEM((ROWS, DIM), jnp.float32),
            pltpu.SemaphoreType.DMA,
        ],
        compiler_params=pltpu.CompilerParams(use_tc_tiling_on_sc=False),
    )(_gather_body)
    return run(table, idx)


def kernel(x, emb_weight, lora_A, lora_B):
    table = _fused_table(emb_weight, lora_A.T, lora_B)
    idx = x.reshape(NW, N_CHUNKS, CHUNK).astype(jnp.int32)
    out = _gather(table, idx)
    return out.reshape(4096, 200, DIM)
